# baseline (speedup 1.0000x reference)
import functools

import jax
import jax.numpy as jnp
from jax import lax
from jax.experimental import pallas as pl
from jax.experimental.pallas import tpu as pltpu
from jax.experimental.pallas import tpu_sc as plsc

B = 16384
GM, GQ, GP = 256, 128, 128
NC, NS, L = 2, 16, 16
NW = NC * NS
BPW = B // NW
NV = BPW // L
GCH = 128
NG = BPW // GCH

_INV_M = float((GM - 1) / 8000.0)
_G0_M = 0.0
_INV_Q = float((GQ - 1) / 1.5)
_G0_Q = -0.5
_INV_P = float((GP - 1) / (2.1e7 - 1.0e5))
_G0_P = 1.0e5


def _nearest16(v, grid_ref, n, g0, inv):
    t = (v - g0) * inv + 0.5
    j0 = jnp.clip(t.astype(jnp.int32), 0, n - 1)
    jm = jnp.maximum(j0 - 1, 0)
    jp = jnp.minimum(j0 + 1, n - 1)
    gm = plsc.load_gather(grid_ref, [jm])
    gc = plsc.load_gather(grid_ref, [j0])
    gp = plsc.load_gather(grid_ref, [jp])
    dm = jnp.abs(v - gm)
    dc = jnp.abs(v - gc)
    dp = jnp.abs(v - gp)
    best_j = jm
    best_d = dm
    take = dc < best_d
    best_j = jnp.where(take, j0, best_j)
    best_d = jnp.where(take, dc, best_d)
    take = dp < best_d
    best_j = jnp.where(take, jp, best_j)
    return best_j


@functools.partial(
    pl.kernel,
    out_type=jax.ShapeDtypeStruct((B,), jnp.float32),
    mesh=plsc.VectorSubcoreMesh(core_axis_name="c", subcore_axis_name="s",
                                num_cores=NC, num_subcores=NS),
    scratch_types=[
        pltpu.VMEM((BPW,), jnp.float32),
        pltpu.VMEM((BPW,), jnp.float32),
        pltpu.VMEM((BPW,), jnp.float32),
        pltpu.VMEM((GM,), jnp.float32),
        pltpu.VMEM((GQ,), jnp.float32),
        pltpu.VMEM((GP,), jnp.float32),
        pltpu.VMEM((NG, GCH), jnp.int32),
        pltpu.VMEM((BPW,), jnp.float32),
        pltpu.SemaphoreType.DMA,
    ],
    compiler_params=pltpu.CompilerParams(needs_layout_passes=False),
)
def _lut_sc(mf_hbm, q_hbm, p_hbm, gm_hbm, gq_hbm, gp_hbm, chf_hbm, out_hbm,
            mf_v, q_v, p_v, gm_v, gq_v, gp_v, idx_v, val_v, sem):
    wid = lax.axis_index("s") * NC + lax.axis_index("c")
    base = wid * BPW

    pltpu.sync_copy(mf_hbm.at[pl.ds(base, BPW)], mf_v)
    pltpu.sync_copy(q_hbm.at[pl.ds(base, BPW)], q_v)
    pltpu.sync_copy(p_hbm.at[pl.ds(base, BPW)], p_v)
    pltpu.sync_copy(gm_hbm, gm_v)
    pltpu.sync_copy(gq_hbm, gq_v)
    pltpu.sync_copy(gp_hbm, gp_v)

    copies = []
    for g in range(NG):
        for c in range(GCH // L):
            i = g * (GCH // L) + c
            s = pl.ds(i * L, L)
            im = _nearest16(mf_v[s], gm_v, GM, _G0_M, _INV_M)
            iq = _nearest16(q_v[s], gq_v, GQ, _G0_Q, _INV_Q)
            ip = _nearest16(p_v[s] * 1000000.0, gp_v, GP, _G0_P, _INV_P)
            idx_v[g, pl.ds(c * L, L)] = (im * (GQ * GP) + iq * GP) + ip
        copies.append(
            pltpu.async_copy(chf_hbm.at[idx_v.at[g]],
                             val_v.at[pl.ds(g * GCH, GCH)], sem))

    for g in range(NG):
        copies[g].wait()
        for c in range(GCH // L):
            s = pl.ds(g * GCH + c * L, L)
            val_v[s] = val_v[s] / 1000000.0

    pltpu.sync_copy(val_v, out_hbm.at[pl.ds(base, BPW)])


def kernel(mass_flux, quality, pressure, mass_flux_grid, quality_grid,
           pressure_grid, chf):
    return _lut_sc(mass_flux, quality, pressure, mass_flux_grid,
                   quality_grid, pressure_grid, chf.reshape(-1))


# speedup vs baseline: 1.1727x; 1.1727x over previous
import functools

import jax
import jax.numpy as jnp
from jax import lax
from jax.experimental import pallas as pl
from jax.experimental.pallas import tpu as pltpu
from jax.experimental.pallas import tpu_sc as plsc

B = 16384
GM, GQ, GP = 256, 128, 128
NC, NS, L = 2, 16, 16
NW = NC * NS
BPW = B // NW
NV = BPW // L
GCH = 128
NG = BPW // GCH

_INV_M = float((GM - 1) / 8000.0)
_G0_M = 0.0
_INV_Q = float((GQ - 1) / 1.5)
_G0_Q = -0.5
_INV_P = float((GP - 1) / (2.1e7 - 1.0e5))
_G0_P = 1.0e5


def _nearest16(v, grid_ref, n, g0, inv):
    t = (v - g0) * inv + 0.5
    j0 = jnp.clip(t.astype(jnp.int32), 0, n - 1)
    jm = jnp.maximum(j0 - 1, 0)
    jp = jnp.minimum(j0 + 1, n - 1)
    gm = plsc.load_gather(grid_ref, [jm])
    gc = plsc.load_gather(grid_ref, [j0])
    gp = plsc.load_gather(grid_ref, [jp])
    dm = jnp.abs(v - gm)
    dc = jnp.abs(v - gc)
    dp = jnp.abs(v - gp)
    best_j = jm
    best_d = dm
    take = dc < best_d
    best_j = jnp.where(take, j0, best_j)
    best_d = jnp.where(take, dc, best_d)
    take = dp < best_d
    best_j = jnp.where(take, jp, best_j)
    return best_j


@functools.partial(
    pl.kernel,
    out_type=jax.ShapeDtypeStruct((B,), jnp.float32),
    mesh=plsc.VectorSubcoreMesh(core_axis_name="c", subcore_axis_name="s",
                                num_cores=NC, num_subcores=NS),
    scratch_types=[
        pltpu.VMEM((BPW,), jnp.float32),
        pltpu.VMEM((BPW,), jnp.float32),
        pltpu.VMEM((BPW,), jnp.float32),
        pltpu.VMEM((GM,), jnp.float32),
        pltpu.VMEM((GQ,), jnp.float32),
        pltpu.VMEM((GP,), jnp.float32),
        pltpu.VMEM((NG, GCH), jnp.int32),
        pltpu.VMEM((BPW,), jnp.float32),
        pltpu.SemaphoreType.DMA,
    ],
    compiler_params=pltpu.CompilerParams(needs_layout_passes=False),
)
def _lut_sc(mf_hbm, q_hbm, p_hbm, gm_hbm, gq_hbm, gp_hbm, chf_hbm, out_hbm,
            mf_v, q_v, p_v, gm_v, gq_v, gp_v, idx_v, val_v, sem):
    wid = lax.axis_index("s") * NC + lax.axis_index("c")
    base = wid * BPW

    in_cps = [
        pltpu.async_copy(mf_hbm.at[pl.ds(base, BPW)], mf_v, sem),
        pltpu.async_copy(q_hbm.at[pl.ds(base, BPW)], q_v, sem),
        pltpu.async_copy(p_hbm.at[pl.ds(base, BPW)], p_v, sem),
        pltpu.async_copy(gm_hbm, gm_v, sem),
        pltpu.async_copy(gq_hbm, gq_v, sem),
        pltpu.async_copy(gp_hbm, gp_v, sem),
    ]
    for cp in in_cps:
        cp.wait()

    gcps = []
    for g in range(NG):
        def chunk(c, carry, g=g):
            s = pl.ds((g * (GCH // L) + c) * L, L)
            im = _nearest16(mf_v[s], gm_v, GM, _G0_M, _INV_M)
            iq = _nearest16(q_v[s], gq_v, GQ, _G0_Q, _INV_Q)
            ip = _nearest16(p_v[s] * 1000000.0, gp_v, GP, _G0_P, _INV_P)
            idx_v[g, pl.ds(c * L, L)] = (im * (GQ * GP) + iq * GP) + ip
            return carry

        lax.fori_loop(0, GCH // L, chunk, 0, unroll=2)
        gcps.append(
            pltpu.async_copy(chf_hbm.at[idx_v.at[g]],
                             val_v.at[pl.ds(g * GCH, GCH)], sem))

    for cp in gcps:
        cp.wait()

    def scale(i, carry):
        s = pl.ds(i * L, L)
        val_v[s] = val_v[s] / 1000000.0
        return carry

    lax.fori_loop(0, NV, scale, 0, unroll=4)
    pltpu.sync_copy(val_v, out_hbm.at[pl.ds(base, BPW)])


def kernel(mass_flux, quality, pressure, mass_flux_grid, quality_grid,
           pressure_grid, chf):
    return _lut_sc(mass_flux, quality, pressure, mass_flux_grid,
                   quality_grid, pressure_grid, chf.reshape(-1))
